# Initial kernel scaffold; baseline (speedup 1.0000x reference)
import jax
import jax.numpy as jnp
from jax.experimental import pallas as pl


def kernel(x, conv_w, conv_b, cls_w, cls_b, bbox_w, bbox_b):
    raise NotImplementedError("write your pallas kernel here")


# speedup vs baseline: 43.2695x; 43.2695x over previous
import functools

import jax
import jax.numpy as jnp
import numpy as np
from jax import lax
from jax.experimental import pallas as pl
from jax.experimental.pallas import tpu as pltpu

_H = 24
_W = 24
_NPOS = _H * _W
_NA = 9
_N = _NPOS * _NA
_ROWS = 48
_NPAD = _ROWS * 128
_CIN = 2048
_COUT = 512
_KCH = 2048
_KTOT = _CIN * 9


def _anchor_consts():
    scales = [8.0, 16.0, 32.0]
    ratios = [0.5, 1.0, 2.0]
    rows = []
    for s in scales:
        for r in ratios:
            w = s * np.sqrt(r)
            h = s / np.sqrt(r)
            rows.append([-w / 2.0, -h / 2.0, w / 2.0, h / 2.0])
    base = np.array(rows, dtype=np.float32)
    widths = base[:, 2] - base[:, 0]
    heights = base[:, 3] - base[:, 1]
    ctr_x = base[:, 0] + np.float32(0.5) * widths
    ctr_y = base[:, 1] + np.float32(0.5) * heights
    return np.concatenate([widths, heights, ctr_x, ctr_y]).reshape(36, 1)


_ANC = _anchor_consts()


def _trunk_kernel(wc_ref, a_ref, wh_ref, cb_ref, hb_ref, anc_ref, out_ref, acc_ref):
    k = pl.program_id(0)

    @pl.when(k == 0)
    def _init():
        acc_ref[:] = jnp.zeros_like(acc_ref)

    acc_ref[:] += jnp.dot(wc_ref[:], a_ref[:],
                          preferred_element_type=jnp.float32)

    @pl.when(k == pl.num_programs(0) - 1)
    def _epilogue():
        h = jnp.maximum(acc_ref[:] + cb_ref[:], 0.0)
        z = jnp.dot(wh_ref[:], h,
                    preferred_element_type=jnp.float32) + hb_ref[:]
        za = z[0:9]
        zb = z[9:18]
        m = jnp.maximum(za, zb)
        ea = jnp.exp(za - m)
        eb = jnp.exp(zb - m)
        s = eb / (ea + eb)
        dx = z[18:27]
        dy = z[27:36]
        dw = z[36:45]
        dh = z[45:54]
        aw = anc_ref[0:9]
        ah = anc_ref[9:18]
        acx = anc_ref[18:27]
        acy = anc_ref[27:36]
        pcx = acx + dx * aw
        pcy = acy + dy * ah
        pw = jnp.exp(dw) * aw
        ph = jnp.exp(dh) * ah
        x1 = pcx - 0.5 * pw
        y1 = pcy - 0.5 * ph
        x2 = pcx + 0.5 * pw
        y2 = pcy + 0.5 * ph
        out_ref[:] = jnp.concatenate([s, x1, y1, x2, y2], axis=0)


def _nms_kernel(s_ref, x1_ref, y1_ref, x2_ref, y2_ref,
                os_ref, ox1_ref, oy1_ref, ox2_ref, oy2_ref):
    os_ref[:] = jnp.zeros_like(os_ref)
    ox1_ref[:] = jnp.zeros_like(ox1_ref)
    oy1_ref[:] = jnp.zeros_like(oy1_ref)
    ox2_ref[:] = jnp.zeros_like(ox2_ref)
    oy2_ref[:] = jnp.zeros_like(oy2_ref)

    s = s_ref[:]
    x1 = x1_ref[:]
    y1 = y1_ref[:]
    x2 = x2_ref[:]
    y2 = y2_ref[:]
    rows = lax.broadcasted_iota(jnp.int32, (_ROWS, 128), 0)
    cols = lax.broadcasted_iota(jnp.int32, (_ROWS, 128), 1)
    idx = rows * 128 + cols
    valid = idx < _N
    areas = jnp.maximum(x2 - x1, 0.0) * jnp.maximum(y2 - y1, 0.0)
    neg = jnp.float32(-jnp.inf)
    big = jnp.int32(2 ** 30)

    def cond(alive_f):
        return jnp.max(alive_f) > 0.0

    def body(alive_f):
        alive = alive_f > 0.0
        sm = jnp.where(alive, s, neg)
        mval = jnp.max(sm)
        cand = alive & (s == mval)
        fidx = jnp.min(jnp.where(cand, idx, big))
        rank = jnp.sum(jnp.where(
            valid & ((s > mval) | ((s == mval) & (idx < fidx))),
            jnp.int32(1), jnp.int32(0)))
        fsel = idx == fidx
        zero = jnp.float32(0.0)
        bx1 = jnp.sum(jnp.where(fsel, x1, zero))
        by1 = jnp.sum(jnp.where(fsel, y1, zero))
        bx2 = jnp.sum(jnp.where(fsel, x2, zero))
        by2 = jnp.sum(jnp.where(fsel, y2, zero))
        farea = jnp.maximum(bx2 - bx1, 0.0) * jnp.maximum(by2 - by1, 0.0)
        xx1 = jnp.maximum(x1, bx1)
        yy1 = jnp.maximum(y1, by1)
        xx2 = jnp.minimum(x2, bx2)
        yy2 = jnp.minimum(y2, by2)
        inter = jnp.maximum(xx2 - xx1, 0.0) * jnp.maximum(yy2 - yy1, 0.0)
        union = (farea + areas) - inter
        iou = inter / jnp.maximum(union, 1e-9)
        suppress = iou > 0.5
        hit = idx == rank
        os_ref[:] = jnp.where(hit, mval, os_ref[:])
        ox1_ref[:] = jnp.where(hit, bx1, ox1_ref[:])
        oy1_ref[:] = jnp.where(hit, by1, oy1_ref[:])
        ox2_ref[:] = jnp.where(hit, bx2, ox2_ref[:])
        oy2_ref[:] = jnp.where(hit, by2, oy2_ref[:])
        return jnp.where(alive & (~suppress) & (~fsel), 1.0, 0.0).astype(jnp.float32)

    lax.while_loop(cond, body, jnp.where(valid, 1.0, 0.0).astype(jnp.float32))


def _pad48(v):
    return jnp.pad(v, (0, _NPAD - _N)).reshape(_ROWS, 128)


@jax.jit
def kernel(x, conv_w, conv_b, cls_w, cls_b, bbox_w, bbox_b):
    xp = jnp.pad(x[0], ((0, 0), (1, 1), (1, 1)))
    cols = []
    for dyx in range(9):
        dy, dxo = divmod(dyx, 3)
        cols.append(xp[:, dy:dy + _H, dxo:dxo + _W].reshape(_CIN, _NPOS))
    a_mat = jnp.concatenate(cols, axis=0)
    wc = jnp.transpose(conv_w, (0, 2, 3, 1)).reshape(_COUT, _KTOT)
    perm = np.concatenate([np.arange(j, 36, 4) for j in range(4)])
    wh = jnp.concatenate([cls_w[:, :, 0, 0], bbox_w[perm, :, 0, 0]], axis=0)
    hb = jnp.concatenate([cls_b, bbox_b[perm]]).reshape(54, 1)
    cb = conv_b.reshape(_COUT, 1)
    anc = jnp.asarray(_ANC)

    nsteps = _KTOT // _KCH
    plane = pl.pallas_call(
        _trunk_kernel,
        grid=(nsteps,),
        in_specs=[
            pl.BlockSpec((_COUT, _KCH), lambda k: (0, k)),
            pl.BlockSpec((_KCH, _NPOS), lambda k: (k, 0)),
            pl.BlockSpec((54, _COUT), lambda k: (0, 0)),
            pl.BlockSpec((_COUT, 1), lambda k: (0, 0)),
            pl.BlockSpec((54, 1), lambda k: (0, 0)),
            pl.BlockSpec((36, 1), lambda k: (0, 0)),
        ],
        out_specs=pl.BlockSpec((45, _NPOS), lambda k: (0, 0)),
        out_shape=jax.ShapeDtypeStruct((45, _NPOS), jnp.float32),
        scratch_shapes=[pltpu.VMEM((_COUT, _NPOS), jnp.float32)],
    )(wc, a_mat, wh, cb, hb, anc)

    s_flat = plane[0:9].reshape(-1)
    x1_flat = plane[9:18].T.reshape(-1)
    y1_flat = plane[18:27].T.reshape(-1)
    x2_flat = plane[27:36].T.reshape(-1)
    y2_flat = plane[36:45].T.reshape(-1)

    outs = pl.pallas_call(
        _nms_kernel,
        out_shape=[jax.ShapeDtypeStruct((_ROWS, 128), jnp.float32)] * 5,
    )(_pad48(s_flat), _pad48(x1_flat), _pad48(y1_flat),
      _pad48(x2_flat), _pad48(y2_flat))

    fs = outs[0].reshape(-1)[:_N]
    fb = jnp.stack([outs[1].reshape(-1)[:_N], outs[2].reshape(-1)[:_N],
                    outs[3].reshape(-1)[:_N], outs[4].reshape(-1)[:_N]], axis=1)
    return fb, fs
